# baseline (speedup 1.0000x reference)
import functools

import jax
import jax.numpy as jnp
from jax import lax
from jax.experimental import pallas as pl
from jax.experimental.pallas import tpu as pltpu
from jax.experimental.pallas import tpu_sc as plsc

_NC = 2
_NS = 16
_L = 16
_NW = _NC * _NS

_B, _N, _K = 64, 2048, 256
_BPW = _B // _NW
_CHUNK = 128
_NCHUNK = _N // _CHUNK


def _presence_counts(psi):
  mesh = plsc.VectorSubcoreMesh(core_axis_name="c", subcore_axis_name="s")

  @functools.partial(
      pl.kernel,
      out_type=jax.ShapeDtypeStruct((_NW * _K,), jnp.float32),
      mesh=mesh,
      scratch_types=[
          pltpu.VMEM((2, _CHUNK, _K), jnp.float32),
          pltpu.VMEM((_K,), jnp.float32),
          pltpu.VMEM((_K,), jnp.float32),
          pltpu.VMEM((_L * 17,), jnp.float32),
          pltpu.VMEM((_L * 17,), jnp.int32),
          pltpu.SemaphoreType.DMA,
          pltpu.SemaphoreType.DMA,
      ],
      compiler_params=pltpu.CompilerParams(
          use_tc_tiling_on_sc=True, needs_layout_passes=False),
  )
  def sc_kernel(psi_hbm, out_hbm, buf, presence, counts, tv, tk, sem0, sem1):
    wid = lax.axis_index("s") * _NC + lax.axis_index("c")
    lanes = lax.iota(jnp.int32, _L)
    zeros = jnp.zeros((_L,), jnp.float32)
    ones = jnp.ones((_L,), jnp.float32)
    lanes17 = lanes * 17

    for j in range(_K // _L):
      counts[pl.ds(j * _L, _L)] = zeros

    def copy_chunk(b, c, parity):
      return pltpu.make_async_copy(
          psi_hbm.at[b, pl.ds(c * _CHUNK, _CHUNK)],
          buf.at[parity],
          sem0 if parity == 0 else sem1,
      )

    def process_chunk(parity):
      def group_body(g, _):
        rowbase = g * _L

        def row_body(rr, _):
          r = rowbase + rr
          nch = 4
          span = _K // _L // nch
          cms = [jnp.full((_L,), -jnp.inf, jnp.float32) for _ in range(nch)]
          cts = [jnp.zeros((_L,), jnp.int32) for _ in range(nch)]
          for t in range(_K // _L):
            c = t // span
            v = buf[parity, r, pl.ds(t * _L, _L)]
            m = v > cms[c]
            cms[c] = jnp.where(m, v, cms[c])
            cts[c] = jnp.where(m, t, cts[c])
          pairs = [(cms[c], cts[c] * _L + lanes) for c in range(nch)]
          while len(pairs) > 1:
            nxtp = []
            for i in range(0, len(pairs), 2):
              (av, ak), (bv, bk) = pairs[i], pairs[i + 1]
              rp = (bv > av) | ((bv == av) & (bk < ak))
              nxtp.append((jnp.where(rp, bv, av), jnp.where(rp, bk, ak)))
            pairs = nxtp
          curmax, kvec = pairs[0]
          tidx = lanes17 + rr
          plsc.store_scatter(tv, [tidx], curmax)
          plsc.store_scatter(tk, [tidx], kvec)
          return 0

        lax.fori_loop(0, _L, row_body, 0, unroll=4)

        cur = []
        for l in range(_L):
          gi = lanes + l * 17
          cur.append((plsc.load_gather(tv, [gi]), plsc.load_gather(tk, [gi])))
        while len(cur) > 1:
          nxt = []
          for i in range(0, len(cur), 2):
            (av, ak), (bv, bk) = cur[i], cur[i + 1]
            repl = (bv > av) | ((bv == av) & (bk < ak))
            nxt.append((jnp.where(repl, bv, av), jnp.where(repl, bk, ak)))
          cur = nxt
        _, bestk = cur[0]
        plsc.store_scatter(presence, [bestk], ones)
        return 0

      lax.fori_loop(0, _CHUNK // _L, group_body, 0)

    for bi in range(_BPW):
      b = wid * _BPW + bi
      for j in range(_K // _L):
        presence[pl.ds(j * _L, _L)] = zeros
      copy_chunk(b, 0, 0).start()
      copy_chunk(b, 1, 1).start()

      def pair(cp, _):
        base = cp * 2
        copy_chunk(b, base, 0).wait()
        process_chunk(0)

        @pl.when(base + 2 < _NCHUNK)
        def _():
          copy_chunk(b, base + 2, 0).start()

        copy_chunk(b, base + 1, 1).wait()
        process_chunk(1)

        @pl.when(base + 3 < _NCHUNK)
        def _():
          copy_chunk(b, base + 3, 1).start()

        return 0

      lax.fori_loop(0, _NCHUNK // 2, pair, 0)

      for j in range(_K // _L):
        sl = pl.ds(j * _L, _L)
        counts[sl] = counts[sl] + presence[sl]

    pltpu.sync_copy(counts, out_hbm.at[pl.ds(wid * _K, _K)])

  return sc_kernel(psi)


def _loss_from_partials(partials):
  def tc_kernel(p_ref, o_ref):
    x = p_ref[...].reshape(_NW, _K)
    counts = jnp.sum(x, axis=0)
    mean = jnp.sum(counts) / _K
    d = counts - mean
    o_ref[0, 0] = jnp.sum(d * d) / _K

  return pl.pallas_call(
      tc_kernel,
      out_shape=jax.ShapeDtypeStruct((1, 1), jnp.float32),
      out_specs=pl.BlockSpec(memory_space=pltpu.SMEM),
  )(partials)


@jax.jit
def kernel(psi):
  partials = _presence_counts(psi)
  loss = _loss_from_partials(partials)
  return loss[0, 0]


# speedup vs baseline: 1.1355x; 1.1355x over previous
import functools

import jax
import jax.numpy as jnp
from jax import lax
from jax.experimental import pallas as pl
from jax.experimental.pallas import tpu as pltpu
from jax.experimental.pallas import tpu_sc as plsc

_NC = 2
_NS = 16
_L = 16
_NW = _NC * _NS

_B, _N, _K = 64, 2048, 256
_BPW = _B // _NW
_CHUNK = 128
_NCHUNK = _N // _CHUNK


def _presence_counts(psi):
  mesh = plsc.VectorSubcoreMesh(core_axis_name="c", subcore_axis_name="s")

  @functools.partial(
      pl.kernel,
      out_type=jax.ShapeDtypeStruct((_NW * _K,), jnp.float32),
      mesh=mesh,
      scratch_types=[
          pltpu.VMEM((2, _CHUNK, _K), jnp.float32),
          pltpu.VMEM((_K,), jnp.float32),
          pltpu.VMEM((_K,), jnp.float32),
          pltpu.VMEM((_L * 17,), jnp.float32),
          pltpu.VMEM((_L * 17,), jnp.int32),
          pltpu.SemaphoreType.DMA,
          pltpu.SemaphoreType.DMA,
      ],
      compiler_params=pltpu.CompilerParams(
          use_tc_tiling_on_sc=True, needs_layout_passes=False),
  )
  def sc_kernel(psi_hbm, out_hbm, buf, presence, counts, tv, tk, sem0, sem1):
    wid = lax.axis_index("s") * _NC + lax.axis_index("c")
    lanes = lax.iota(jnp.int32, _L)
    zeros = jnp.zeros((_L,), jnp.float32)
    ones = jnp.ones((_L,), jnp.float32)
    lanes17 = lanes * 17

    for j in range(_K // _L):
      counts[pl.ds(j * _L, _L)] = zeros

    def copy_chunk(b, c, parity):
      return pltpu.make_async_copy(
          psi_hbm.at[b, pl.ds(c * _CHUNK, _CHUNK)],
          buf.at[parity],
          sem0 if parity == 0 else sem1,
      )

    def process_chunk(parity):
      def group_body(g, _):
        rowbase = g * _L

        def row_body(rr, _):
          r = rowbase + rr
          nch = 2
          span = _K // _L // nch
          cms = [jnp.full((_L,), -jnp.inf, jnp.float32) for _ in range(nch)]
          cts = [jnp.zeros((_L,), jnp.int32) for _ in range(nch)]
          for t in range(_K // _L):
            c = t // span
            v = buf[parity, r, pl.ds(t * _L, _L)]
            m = v > cms[c]
            cms[c] = jnp.where(m, v, cms[c])
            cts[c] = jnp.where(m, t, cts[c])
          pairs = [(cms[c], cts[c] * _L + lanes) for c in range(nch)]
          while len(pairs) > 1:
            nxtp = []
            for i in range(0, len(pairs), 2):
              (av, ak), (bv, bk) = pairs[i], pairs[i + 1]
              rp = (bv > av) | ((bv == av) & (bk < ak))
              nxtp.append((jnp.where(rp, bv, av), jnp.where(rp, bk, ak)))
            pairs = nxtp
          curmax, kvec = pairs[0]
          tidx = lanes17 + rr
          plsc.store_scatter(tv, [tidx], curmax)
          plsc.store_scatter(tk, [tidx], kvec)
          return 0

        lax.fori_loop(0, _L, row_body, 0, unroll=4)

        cur = []
        for l in range(_L):
          gi = lanes + l * 17
          cur.append((plsc.load_gather(tv, [gi]), plsc.load_gather(tk, [gi])))
        while len(cur) > 1:
          nxt = []
          for i in range(0, len(cur), 2):
            (av, ak), (bv, bk) = cur[i], cur[i + 1]
            repl = (bv > av) | ((bv == av) & (bk < ak))
            nxt.append((jnp.where(repl, bv, av), jnp.where(repl, bk, ak)))
          cur = nxt
        _, bestk = cur[0]
        plsc.store_scatter(presence, [bestk], ones)
        return 0

      lax.fori_loop(0, _CHUNK // _L, group_body, 0)

    for bi in range(_BPW):
      b = wid * _BPW + bi
      for j in range(_K // _L):
        presence[pl.ds(j * _L, _L)] = zeros
      copy_chunk(b, 0, 0).start()
      copy_chunk(b, 1, 1).start()

      def pair(cp, _):
        base = cp * 2
        copy_chunk(b, base, 0).wait()
        process_chunk(0)

        @pl.when(base + 2 < _NCHUNK)
        def _():
          copy_chunk(b, base + 2, 0).start()

        copy_chunk(b, base + 1, 1).wait()
        process_chunk(1)

        @pl.when(base + 3 < _NCHUNK)
        def _():
          copy_chunk(b, base + 3, 1).start()

        return 0

      lax.fori_loop(0, _NCHUNK // 2, pair, 0)

      for j in range(_K // _L):
        sl = pl.ds(j * _L, _L)
        counts[sl] = counts[sl] + presence[sl]

    pltpu.sync_copy(counts, out_hbm.at[pl.ds(wid * _K, _K)])

  return sc_kernel(psi)


def _loss_from_partials(partials):
  def tc_kernel(p_ref, o_ref):
    x = p_ref[...].reshape(_NW, _K)
    counts = jnp.sum(x, axis=0)
    mean = jnp.sum(counts) / _K
    d = counts - mean
    o_ref[0, 0] = jnp.sum(d * d) / _K

  return pl.pallas_call(
      tc_kernel,
      out_shape=jax.ShapeDtypeStruct((1, 1), jnp.float32),
      out_specs=pl.BlockSpec(memory_space=pltpu.SMEM),
  )(partials)


@jax.jit
def kernel(psi):
  partials = _presence_counts(psi)
  loss = _loss_from_partials(partials)
  return loss[0, 0]


# speedup vs baseline: 1.1363x; 1.0008x over previous
import functools

import jax
import jax.numpy as jnp
from jax import lax
from jax.experimental import pallas as pl
from jax.experimental.pallas import tpu as pltpu
from jax.experimental.pallas import tpu_sc as plsc

_NC = 2
_NS = 16
_L = 16
_NW = _NC * _NS

_B, _N, _K = 64, 2048, 256
_BPW = _B // _NW
_CHUNK = 64
_NCHUNK = _N // _CHUNK


def _presence_counts(psi):
  mesh = plsc.VectorSubcoreMesh(core_axis_name="c", subcore_axis_name="s")

  @functools.partial(
      pl.kernel,
      out_type=jax.ShapeDtypeStruct((_NW * _K,), jnp.float32),
      mesh=mesh,
      scratch_types=[
          pltpu.VMEM((2, _CHUNK, _K), jnp.float32),
          pltpu.VMEM((_K,), jnp.float32),
          pltpu.VMEM((_K,), jnp.float32),
          pltpu.VMEM((_L * 17,), jnp.float32),
          pltpu.VMEM((_L * 17,), jnp.int32),
          pltpu.SemaphoreType.DMA,
          pltpu.SemaphoreType.DMA,
      ],
      compiler_params=pltpu.CompilerParams(
          use_tc_tiling_on_sc=True, needs_layout_passes=False),
  )
  def sc_kernel(psi_hbm, out_hbm, buf, presence, counts, tv, tk, sem0, sem1):
    wid = lax.axis_index("s") * _NC + lax.axis_index("c")
    lanes = lax.iota(jnp.int32, _L)
    zeros = jnp.zeros((_L,), jnp.float32)
    ones = jnp.ones((_L,), jnp.float32)
    lanes17 = lanes * 17

    for j in range(_K // _L):
      counts[pl.ds(j * _L, _L)] = zeros

    def copy_chunk(b, c, parity):
      return pltpu.make_async_copy(
          psi_hbm.at[b, pl.ds(c * _CHUNK, _CHUNK)],
          buf.at[parity],
          sem0 if parity == 0 else sem1,
      )

    def process_chunk(parity):
      def group_body(g, _):
        rowbase = g * _L

        def row_body(rr, _):
          r = rowbase + rr
          nch = 2
          span = _K // _L // nch
          cms = [jnp.full((_L,), -jnp.inf, jnp.float32) for _ in range(nch)]
          cts = [jnp.zeros((_L,), jnp.int32) for _ in range(nch)]
          for t in range(_K // _L):
            c = t // span
            v = buf[parity, r, pl.ds(t * _L, _L)]
            m = v > cms[c]
            cms[c] = jnp.where(m, v, cms[c])
            cts[c] = jnp.where(m, t, cts[c])
          pairs = [(cms[c], cts[c] * _L + lanes) for c in range(nch)]
          while len(pairs) > 1:
            nxtp = []
            for i in range(0, len(pairs), 2):
              (av, ak), (bv, bk) = pairs[i], pairs[i + 1]
              rp = (bv > av) | ((bv == av) & (bk < ak))
              nxtp.append((jnp.where(rp, bv, av), jnp.where(rp, bk, ak)))
            pairs = nxtp
          curmax, kvec = pairs[0]
          tidx = lanes17 + rr
          plsc.store_scatter(tv, [tidx], curmax)
          plsc.store_scatter(tk, [tidx], kvec)
          return 0

        lax.fori_loop(0, _L, row_body, 0, unroll=4)

        cur = []
        for l in range(_L):
          gi = lanes + l * 17
          cur.append((plsc.load_gather(tv, [gi]), plsc.load_gather(tk, [gi])))
        while len(cur) > 1:
          nxt = []
          for i in range(0, len(cur), 2):
            (av, ak), (bv, bk) = cur[i], cur[i + 1]
            repl = (bv > av) | ((bv == av) & (bk < ak))
            nxt.append((jnp.where(repl, bv, av), jnp.where(repl, bk, ak)))
          cur = nxt
        _, bestk = cur[0]
        plsc.store_scatter(presence, [bestk], ones)
        return 0

      lax.fori_loop(0, _CHUNK // _L, group_body, 0)

    for bi in range(_BPW):
      b = wid * _BPW + bi
      for j in range(_K // _L):
        presence[pl.ds(j * _L, _L)] = zeros
      copy_chunk(b, 0, 0).start()
      copy_chunk(b, 1, 1).start()

      def pair(cp, _):
        base = cp * 2
        copy_chunk(b, base, 0).wait()
        process_chunk(0)

        @pl.when(base + 2 < _NCHUNK)
        def _():
          copy_chunk(b, base + 2, 0).start()

        copy_chunk(b, base + 1, 1).wait()
        process_chunk(1)

        @pl.when(base + 3 < _NCHUNK)
        def _():
          copy_chunk(b, base + 3, 1).start()

        return 0

      lax.fori_loop(0, _NCHUNK // 2, pair, 0)

      for j in range(_K // _L):
        sl = pl.ds(j * _L, _L)
        counts[sl] = counts[sl] + presence[sl]

    pltpu.sync_copy(counts, out_hbm.at[pl.ds(wid * _K, _K)])

  return sc_kernel(psi)


def _loss_from_partials(partials):
  def tc_kernel(p_ref, o_ref):
    x = p_ref[...].reshape(_NW, _K)
    counts = jnp.sum(x, axis=0)
    mean = jnp.sum(counts) / _K
    d = counts - mean
    o_ref[0, 0] = jnp.sum(d * d) / _K

  return pl.pallas_call(
      tc_kernel,
      out_shape=jax.ShapeDtypeStruct((1, 1), jnp.float32),
      out_specs=pl.BlockSpec(memory_space=pltpu.SMEM),
  )(partials)


@jax.jit
def kernel(psi):
  partials = _presence_counts(psi)
  loss = _loss_from_partials(partials)
  return loss[0, 0]


# speedup vs baseline: 1.2629x; 1.1114x over previous
import functools

import jax
import jax.numpy as jnp
from jax import lax
from jax.experimental import pallas as pl
from jax.experimental.pallas import tpu as pltpu
from jax.experimental.pallas import tpu_sc as plsc

_NC = 2
_NS = 16
_L = 16
_NW = _NC * _NS

_B, _N, _K = 64, 2048, 256
_BPW = _B // _NW
_CHUNK = 64
_NCHUNK = _N // _CHUNK


def _presence_counts(psi):
  mesh = plsc.VectorSubcoreMesh(core_axis_name="c", subcore_axis_name="s")

  @functools.partial(
      pl.kernel,
      out_type=jax.ShapeDtypeStruct((_NW * _K,), jnp.float32),
      mesh=mesh,
      scratch_types=[
          pltpu.VMEM((2, _CHUNK, _K), jnp.float32),
          pltpu.VMEM((_K,), jnp.float32),
          pltpu.VMEM((_K,), jnp.float32),
          pltpu.VMEM((_L * 17,), jnp.float32),
          pltpu.VMEM((_L * 17,), jnp.int32),
          pltpu.SemaphoreType.DMA,
          pltpu.SemaphoreType.DMA,
      ],
      compiler_params=pltpu.CompilerParams(
          use_tc_tiling_on_sc=True, needs_layout_passes=False),
  )
  def sc_kernel(psi_hbm, out_hbm, buf, presence, counts, tv, tk, sem0, sem1):
    wid = lax.axis_index("s") * _NC + lax.axis_index("c")
    lanes = lax.iota(jnp.int32, _L)
    zeros = jnp.zeros((_L,), jnp.float32)
    ones = jnp.ones((_L,), jnp.float32)
    lanes17 = lanes * 17

    for j in range(_K // _L):
      counts[pl.ds(j * _L, _L)] = zeros

    def copy_chunk(b, c, parity):
      return pltpu.make_async_copy(
          psi_hbm.at[b, pl.ds(c * _CHUNK, _CHUNK)],
          buf.at[parity],
          sem0 if parity == 0 else sem1,
      )

    def process_chunk(parity):
      def group_body(g, _):
        rowbase = g * _L

        def row_body(rr):
          r = rowbase + rr
          nch = 2
          span = _K // _L // nch
          cms = [jnp.full((_L,), -jnp.inf, jnp.float32) for _ in range(nch)]
          cts = [jnp.zeros((_L,), jnp.int32) for _ in range(nch)]
          for t in range(_K // _L):
            c = t // span
            v = buf[parity, r, pl.ds(t * _L, _L)]
            m = v > cms[c]
            cms[c] = jnp.where(m, v, cms[c])
            cts[c] = jnp.where(m, t, cts[c])
          pairs = [(cms[c], cts[c] * _L + lanes) for c in range(nch)]
          while len(pairs) > 1:
            nxtp = []
            for i in range(0, len(pairs), 2):
              (av, ak), (bv, bk) = pairs[i], pairs[i + 1]
              rp = (bv > av) | ((bv == av) & (bk < ak))
              nxtp.append((jnp.where(rp, bv, av), jnp.where(rp, bk, ak)))
            pairs = nxtp
          curmax, kvec = pairs[0]
          tidx = lanes17 + rr
          plsc.store_scatter(tv, [tidx], curmax)
          plsc.store_scatter(tk, [tidx], kvec)

        plsc.parallel_loop(0, _L, unroll=4)(row_body)

        cur = []
        for l in range(_L):
          gi = lanes + l * 17
          cur.append((plsc.load_gather(tv, [gi]), plsc.load_gather(tk, [gi])))
        while len(cur) > 1:
          nxt = []
          for i in range(0, len(cur), 2):
            (av, ak), (bv, bk) = cur[i], cur[i + 1]
            repl = (bv > av) | ((bv == av) & (bk < ak))
            nxt.append((jnp.where(repl, bv, av), jnp.where(repl, bk, ak)))
          cur = nxt
        _, bestk = cur[0]
        plsc.store_scatter(presence, [bestk], ones)
        return 0

      lax.fori_loop(0, _CHUNK // _L, group_body, 0)

    for bi in range(_BPW):
      b = wid * _BPW + bi
      for j in range(_K // _L):
        presence[pl.ds(j * _L, _L)] = zeros
      copy_chunk(b, 0, 0).start()
      copy_chunk(b, 1, 1).start()

      def pair(cp, _):
        base = cp * 2
        copy_chunk(b, base, 0).wait()
        process_chunk(0)

        @pl.when(base + 2 < _NCHUNK)
        def _():
          copy_chunk(b, base + 2, 0).start()

        copy_chunk(b, base + 1, 1).wait()
        process_chunk(1)

        @pl.when(base + 3 < _NCHUNK)
        def _():
          copy_chunk(b, base + 3, 1).start()

        return 0

      lax.fori_loop(0, _NCHUNK // 2, pair, 0)

      for j in range(_K // _L):
        sl = pl.ds(j * _L, _L)
        counts[sl] = counts[sl] + presence[sl]

    pltpu.sync_copy(counts, out_hbm.at[pl.ds(wid * _K, _K)])

  return sc_kernel(psi)


def _loss_from_partials(partials):
  def tc_kernel(p_ref, o_ref):
    x = p_ref[...].reshape(_NW, _K)
    counts = jnp.sum(x, axis=0)
    mean = jnp.sum(counts) / _K
    d = counts - mean
    o_ref[0, 0] = jnp.sum(d * d) / _K

  return pl.pallas_call(
      tc_kernel,
      out_shape=jax.ShapeDtypeStruct((1, 1), jnp.float32),
      out_specs=pl.BlockSpec(memory_space=pltpu.SMEM),
  )(partials)


@jax.jit
def kernel(psi):
  partials = _presence_counts(psi)
  loss = _loss_from_partials(partials)
  return loss[0, 0]
